# baseline (speedup 1.0000x reference)
import functools

import jax
import jax.numpy as jnp
from jax import lax
from jax.experimental import pallas as pl
from jax.experimental.pallas import tpu as pltpu
from jax.experimental.pallas import tpu_sc as plsc

N_ROWS_TBL = 100000
N_EMB = 64
BATCH = 16384

NC = 2
NS = 16
NW = NC * NS
B_PER_W = BATCH // NW
N_CHUNK = B_PER_W // 128


def _mfnet_sc(task2d, cons2d, te_hbm, ce_hbm, tb_hbm, cb_hbm, wb_hbm, out_hbm,
              idx_t, idx_c, te_rows, ce_rows, tb_v, cb_v, wb_v, out_v, pbuf, sem):
    wid = lax.axis_index("s") * NC + lax.axis_index("c")
    base = wid * B_PER_W

    pltpu.sync_copy(task2d.at[pl.ds(wid * N_CHUNK, N_CHUNK)], idx_t)
    pltpu.sync_copy(cons2d.at[pl.ds(wid * N_CHUNK, N_CHUNK)], idx_c)
    pltpu.sync_copy(wb_hbm, wb_v)

    copies = []
    for j in range(N_CHUNK):
        sl = pl.ds(j * 128, 128)
        copies.append(pltpu.async_copy(te_hbm.at[idx_t.at[j]], te_rows.at[sl], sem))
        copies.append(pltpu.async_copy(ce_hbm.at[idx_c.at[j]], ce_rows.at[sl], sem))
        copies.append(pltpu.async_copy(tb_hbm.at[idx_t.at[j]], tb_v.at[sl], sem))
        copies.append(pltpu.async_copy(cb_hbm.at[idx_c.at[j]], cb_v.at[sl], sem))
    for c in copies:
        c.wait()

    wch = [wb_v[pl.ds(k * 16, 16)] for k in range(N_EMB // 16)]
    bd = wb_v[pl.ds(N_EMB, 16)][0]
    w_sc = [wch[k][l] for k in range(N_EMB // 16) for l in range(16)]
    s_tot = functools.reduce(lambda a, b: a + b, w_sc)

    lane = lax.iota(jnp.int32, 16)

    def group_body(g, _):
        r0 = g * 16
        for i in range(16):
            r = r0 + i
            p = te_rows[r, pl.ds(0, 16)] * ce_rows[r, pl.ds(0, 16)] * wch[0]
            for k in range(1, N_EMB // 16):
                p = p + (te_rows[r, pl.ds(k * 16, 16)]
                         * ce_rows[r, pl.ds(k * 16, 16)] * wch[k])
            pbuf[pl.ds(i * 16, 16)] = p
        acc = plsc.load_gather(pbuf, [lane * 16])
        for c in range(1, 16):
            acc = acc + plsc.load_gather(pbuf, [lane * 16 + c])
        tb = tb_v[pl.ds(r0, 16)]
        cb = cb_v[pl.ds(r0, 16)]
        out_v[pl.ds(r0, 16)] = acc + s_tot * (tb + cb) + bd
        return 0

    lax.fori_loop(0, B_PER_W // 16, group_body, 0)

    pltpu.sync_copy(out_v, out_hbm.at[pl.ds(base, B_PER_W)])


@jax.jit
def _mfnet(task2d, cons2d, te, ce, tb, cb, wb):
    mesh = plsc.VectorSubcoreMesh(core_axis_name="c", subcore_axis_name="s")
    f = functools.partial(
        pl.kernel,
        out_type=jax.ShapeDtypeStruct((BATCH,), jnp.float32),
        mesh=mesh,
        compiler_params=pltpu.CompilerParams(needs_layout_passes=False, use_tc_tiling_on_sc=False),
        scratch_types=[
            pltpu.VMEM((N_CHUNK, 128), jnp.int32),
            pltpu.VMEM((N_CHUNK, 128), jnp.int32),
            pltpu.VMEM((B_PER_W, N_EMB), jnp.float32),
            pltpu.VMEM((B_PER_W, N_EMB), jnp.float32),
            pltpu.VMEM((B_PER_W,), jnp.float32),
            pltpu.VMEM((B_PER_W,), jnp.float32),
            pltpu.VMEM((N_EMB + 16,), jnp.float32),
            pltpu.VMEM((B_PER_W,), jnp.float32),
            pltpu.VMEM((256,), jnp.float32),
            pltpu.SemaphoreType.DMA,
        ],
    )(_mfnet_sc)
    return f(task2d, cons2d, te, ce, tb, cb, wb)


def kernel(task, construct, task_emb, construct_emb, task_bias, construct_bias,
           W_dec, b_dec):
    task2d = task.astype(jnp.int32).reshape(BATCH // 128, 128)
    cons2d = construct.astype(jnp.int32).reshape(BATCH // 128, 128)
    tb = task_bias.reshape(N_ROWS_TBL)
    cb = construct_bias.reshape(N_ROWS_TBL)
    wb = jnp.concatenate([W_dec.reshape(N_EMB),
                          jnp.broadcast_to(b_dec, (16,)).astype(jnp.float32)])
    y = _mfnet(task2d, cons2d, task_emb, construct_emb, tb, cb, wb)
    return y.reshape(BATCH, 1)
